# baseline (speedup 1.0000x reference)
import jax
import jax.numpy as jnp
from jax import lax
from jax.experimental import pallas as pl
from jax.experimental.pallas import tpu as pltpu
from jax.experimental.pallas import tpu_sc as plsc

N = 50000
E = 800000
B = 50
H = 64
C = 3
Bp = 64

NC, NS = 2, 16
NW = NC * NS
EW = E // NW
GK = 5000
ROWS_T = N // NS

BLKE = 6400
EB8 = BLKE // 8
EB16 = BLKE // 16
BLKD = 2000

_MESH_KW = dict(core_axis_name="c", subcore_axis_name="s",
                num_cores=NC, num_subcores=NS)


def _leaky(x):
    return jnp.where(x > 0, x, 0.2 * x)


def _dot(a, b):
    return jnp.dot(a, b, preferred_element_type=jnp.float32)


def _sc_gather(coordp, row, col):
    mesh = plsc.VectorSubcoreMesh(**_MESH_KW)

    def body(coordp_hbm, row_hbm, col_hbm, ca_hbm, cb_hbm, idx_v, rows_v, sem):
        wid = lax.axis_index("s") * NC + lax.axis_index("c")
        base = wid * EW
        for j in range(EW // GK):
            off = base + j * GK
            pltpu.sync_copy(row_hbm.at[pl.ds(off, GK)], idx_v)
            pltpu.async_copy(coordp_hbm.at[idx_v], rows_v, sem).wait()
            pltpu.sync_copy(rows_v, ca_hbm.at[pl.ds(off, GK)])
            pltpu.sync_copy(col_hbm.at[pl.ds(off, GK)], idx_v)
            pltpu.async_copy(coordp_hbm.at[idx_v], rows_v, sem).wait()
            pltpu.sync_copy(rows_v, cb_hbm.at[pl.ds(off, GK)])

    out_type = (jax.ShapeDtypeStruct((E, 16), jnp.float32),
                jax.ShapeDtypeStruct((E, 16), jnp.float32))
    return pl.kernel(
        body, out_type=out_type, mesh=mesh,
        compiler_params=pltpu.CompilerParams(use_tc_tiling_on_sc=False),
        scratch_types=[
            pltpu.VMEM((GK,), jnp.int32),
            pltpu.VMEM((GK, 16), jnp.float32),
            pltpu.SemaphoreType.DMA,
        ])(coordp, row, col)


def _edge_mlp(ca, cb, ea, phi_w1, phi_b1, phi_w2, em_w1, em_b1, em_w2, em_b2):
    w1r = phi_w1[:, 0].reshape(1, H)
    w1aT = phi_w1[:, 1:].T
    b1 = phi_b1.reshape(1, H)
    w2T = phi_w2.T
    emw1T = em_w1.T
    emb1 = em_b1.reshape(1, H)
    emw2T = em_w2.T
    emb2 = em_b2.reshape(1, 1)
    grid = E // BLKE

    def body(ca_ref, cb_ref, ea_ref, w1r_ref, w1aT_ref, b1_ref, w2T_ref,
             emw1T_ref, emb1_ref, emw2T_ref, emb2_ref, out_ref):
        d = ca_ref[...] - cb_ref[...]
        ea = ea_ref[...]
        pieces = []
        for k in range(8):
            dk = d[:, 16 * k:16 * k + 16]
            radial = jnp.sum(dk * dk, axis=1, keepdims=True)
            eak = ea[EB8 * k:EB8 * (k + 1), :]
            h1 = radial * w1r_ref[...] + _dot(eak, w1aT_ref[...]) + b1_ref[...]
            f = jnp.tanh(_dot(_leaky(h1), w2T_ref[...]))
            g = _leaky(_dot(f, emw1T_ref[...]) + emb1_ref[...])
            m = _dot(g, emw2T_ref[...]) + emb2_ref[...]
            cidx = lax.broadcasted_iota(jnp.int32, (EB8, 8), 1)
            tk = jnp.where(cidx < 3, dk[:, :8] * m,
                           jnp.where(cidx == 3, 1.0, 0.0))
            pieces.append(tk[:EB16])
            pieces.append(tk[EB16:])
        out_ref[...] = jnp.concatenate(pieces, axis=1)

    const = lambda shape: pl.BlockSpec(shape, lambda i: (0,) * len(shape))
    return pl.pallas_call(
        body,
        grid=(grid,),
        in_specs=[
            pl.BlockSpec((EB8, 128), lambda i: (i, 0)),
            pl.BlockSpec((EB8, 128), lambda i: (i, 0)),
            pl.BlockSpec((BLKE, 4), lambda i: (i, 0)),
            const((1, H)), const((4, H)), const((1, H)), const((H, H)),
            const((H, H)), const((1, H)), const((H, 1)), const((1, 1)),
        ],
        out_specs=pl.BlockSpec((EB16, 128), lambda i: (i, 0)),
        out_shape=jax.ShapeDtypeStruct((E // 16, 128), jnp.float32),
    )(ca, cb, ea, w1r, w1aT, b1, w2T, emw1T, emb1, emw2T, emb2)


def _sc_scatter(trans, row, zrows):
    mesh = plsc.VectorSubcoreMesh(**_MESH_KW)

    def body(trans_hbm, row_hbm, z_hbm, out_hbm, idx_v, tr_v, sem, acc):
        cid = lax.axis_index("c")
        sid = lax.axis_index("s")
        wid = sid * NC + cid
        pltpu.sync_copy(z_hbm, acc.at[pl.ds(sid * ROWS_T, ROWS_T)])
        plsc.subcore_barrier()
        base = wid * EW
        for j in range(EW // GK):
            off = base + j * GK
            pltpu.sync_copy(row_hbm.at[pl.ds(off, GK)], idx_v)
            pltpu.sync_copy(trans_hbm.at[pl.ds(off, GK)], tr_v)
            pltpu.sync_copy(tr_v, acc.at[idx_v], add=True)
        plsc.subcore_barrier()
        pltpu.sync_copy(acc.at[pl.ds(sid * ROWS_T, ROWS_T)],
                        out_hbm.at[cid, pl.ds(sid * ROWS_T, ROWS_T)])

    return pl.kernel(
        body,
        out_type=jax.ShapeDtypeStruct((NC, N, 8), jnp.float32),
        mesh=mesh,
        compiler_params=pltpu.CompilerParams(use_tc_tiling_on_sc=False),
        scratch_types=[
            pltpu.VMEM((GK,), jnp.int32),
            pltpu.VMEM((GK, 8), jnp.float32),
            pltpu.SemaphoreType.DMA,
            pltpu.VMEM_SHARED((N, 8), jnp.float32),
        ])(trans, row, zrows)


def _node_stats(dbT3, coord, vc9p):
    grid = N // BLKD

    def body(dbT_ref, coord_ref, vc9p_ref, out_ref, acc_ref):
        i = pl.program_id(0)

        @pl.when(i == 0)
        def _init():
            acc_ref[...] = jnp.zeros((Bp, 16), jnp.float32)

        dbt = dbT_ref[0]
        onehotT = (lax.broadcasted_iota(jnp.int32, (Bp, BLKD), 0)
                   == dbt).astype(jnp.float32)
        cx = jnp.concatenate(
            [coord_ref[...],
             jnp.ones((BLKD, 1), jnp.float32),
             jnp.zeros((BLKD, 12), jnp.float32)], axis=1)
        acc_ref[...] += _dot(onehotT, cx)

        @pl.when(i == grid - 1)
        def _fin():
            s = acc_ref[...]
            cnt = jnp.maximum(s[:, 3:4], 1.0)
            cm = s[:, 0:3] / cnt
            vc9 = vc9p_ref[...][:, 0:9]
            cm9 = jnp.concatenate(
                [cm[:, 0:1]] * 3 + [cm[:, 1:2]] * 3 + [cm[:, 2:3]] * 3, axis=1)
            A = vc9 - cm9
            cols = []
            for i_ in range(3):
                for j_ in range(3):
                    cols.append(A[:, i_:i_ + 1] * A[:, j_:j_ + 1]
                                + A[:, 3 + i_:4 + i_] * A[:, 3 + j_:4 + j_]
                                + A[:, 6 + i_:7 + i_] * A[:, 6 + j_:7 + j_])
            out_ref[...] = jnp.concatenate(
                [cm, cnt] + cols + [jnp.zeros((Bp, 3), jnp.float32)], axis=1)

    return pl.pallas_call(
        body,
        grid=(grid,),
        in_specs=[
            pl.BlockSpec((1, 1, BLKD), lambda i: (i, 0, 0)),
            pl.BlockSpec((BLKD, 3), lambda i: (i, 0)),
            pl.BlockSpec((Bp, 16), lambda i: (0, 0)),
        ],
        out_specs=pl.BlockSpec((Bp, 16), lambda i: (0, 0)),
        out_shape=jax.ShapeDtypeStruct((Bp, 16), jnp.float32),
        scratch_shapes=[pltpu.VMEM((Bp, 16), jnp.float32)],
    )(dbT3, coord, vc9p)


def _node_main(dbT3, db2, coord, vel, s0, s1, stats, vc9p,
               phivw1T, phivb1, phivw2T,
               rvw1T, rvb1, rvw2T, rvb2,
               vrw1T, vrb1, vrw2T, vrb2,
               cvw1r, cvb1, cvw2T, cvb2):
    grid = N // BLKD

    def body(dbT_ref, db_ref, coord_ref, vel_ref, s0_ref, s1_ref,
             stats_ref, vc9p_ref,
             phivw1T_ref, phivb1_ref, phivw2T_ref,
             rvw1T_ref, rvb1_ref, rvw2T_ref, rvb2_ref,
             vrw1T_ref, vrb1_ref, vrw2T_ref, vrb2_ref,
             cvw1r_ref, cvb1_ref, cvw2T_ref, cvb2_ref,
             coord2_ref, vout_ref, vacc_ref):
        i = pl.program_id(0)

        @pl.when(i == 0)
        def _init():
            vacc_ref[...] = jnp.zeros((Bp, 16), jnp.float32)

        st = stats_ref[...]
        cnt_g = st[:, 3:4]
        mX9 = st[:, 4:13]
        vc9 = vc9p_ref[...][:, 0:9]

        db = db_ref[...]
        onehot = (db == lax.broadcasted_iota(jnp.int32, (BLKD, Bp), 1)
                  ).astype(jnp.float32)
        dbt = dbT_ref[0]
        onehotT = (lax.broadcasted_iota(jnp.int32, (Bp, BLKD), 0)
                   == dbt).astype(jnp.float32)

        co = coord_ref[...]
        vcdb = _dot(onehot, vc9)
        mXdb = _dot(onehot, mX9)
        co9 = jnp.concatenate(
            [co[:, 0:1]] * 3 + [co[:, 1:2]] * 3 + [co[:, 2:3]] * 3, axis=1)
        vcd9 = vcdb - co9
        sq = vcd9 * vcd9
        vrad = jnp.sqrt(sq[:, 0:3] + sq[:, 3:6] + sq[:, 6:9])

        rv_cols, vr_cols = [], []
        for c in range(C):
            Xc = jnp.concatenate(
                [vrad[:, c:c + 1], mXdb[:, c:c + 1],
                 mXdb[:, 3 + c:4 + c], mXdb[:, 6 + c:7 + c]], axis=1)
            h = _leaky(_dot(Xc, phivw1T_ref[...]) + phivb1_ref[...])
            vef = jnp.tanh(_dot(h, phivw2T_ref[...]))
            hr = _leaky(_dot(vef, rvw1T_ref[...]) + rvb1_ref[...])
            rv_cols.append(_dot(hr, rvw2T_ref[...]) + rvb2_ref[...])
            hv = _leaky(_dot(vef, vrw1T_ref[...]) + vrb1_ref[...])
            vr_cols.append(_dot(hv, vrw2T_ref[...]) + vrb2_ref[...])
        rv = jnp.concatenate(rv_cols, axis=1)
        vr = jnp.concatenate(vr_cols, axis=1)

        rv9 = jnp.concatenate([rv] * 3, axis=1)
        prod = vcd9 * rv9
        vterm = -jnp.concatenate(
            [jnp.sum(prod[:, 0:3], axis=1, keepdims=True),
             jnp.sum(prod[:, 3:6], axis=1, keepdims=True),
             jnp.sum(prod[:, 6:9], axis=1, keepdims=True)], axis=1) / 3.0

        vr9 = jnp.concatenate([vr] * 3, axis=1)
        trans2 = vcd9 * vr9

        v = vel_ref[...]
        vnorm = jnp.sqrt(jnp.sum(v * v, axis=1, keepdims=True))
        hc = _leaky(vnorm * cvw1r_ref[...] + cvb1_ref[...])
        cvout = _dot(hc, cvw2T_ref[...]) + cvb2_ref[...]

        es = s0_ref[...] + s1_ref[...]
        emean = es[:, 0:3] / jnp.maximum(es[:, 3:4], 1.0)

        coord2_ref[...] = co + emean + vterm + v * cvout

        tr2p = jnp.concatenate(
            [trans2, jnp.zeros((BLKD, 7), jnp.float32)], axis=1)
        vacc_ref[...] += _dot(onehotT, tr2p)

        @pl.when(i == grid - 1)
        def _fin():
            vout_ref[...] = vc9p_ref[...] + vacc_ref[...] / cnt_g

    const = lambda shape: pl.BlockSpec(shape, lambda i: (0,) * len(shape))
    return pl.pallas_call(
        body,
        grid=(grid,),
        in_specs=[
            pl.BlockSpec((1, 1, BLKD), lambda i: (i, 0, 0)),
            pl.BlockSpec((BLKD, 1), lambda i: (i, 0)),
            pl.BlockSpec((BLKD, 3), lambda i: (i, 0)),
            pl.BlockSpec((BLKD, 3), lambda i: (i, 0)),
            pl.BlockSpec((BLKD, 8), lambda i: (i, 0)),
            pl.BlockSpec((BLKD, 8), lambda i: (i, 0)),
            const((Bp, 16)), const((Bp, 16)),
            const((4, H)), const((1, H)), const((H, H)),
            const((H, H)), const((1, H)), const((H, 1)), const((1, 1)),
            const((H, H)), const((1, H)), const((H, 1)), const((1, 1)),
            const((1, H)), const((1, H)), const((H, 1)), const((1, 1)),
        ],
        out_specs=[
            pl.BlockSpec((BLKD, 3), lambda i: (i, 0)),
            pl.BlockSpec((Bp, 16), lambda i: (0, 0)),
        ],
        out_shape=[
            jax.ShapeDtypeStruct((N, 3), jnp.float32),
            jax.ShapeDtypeStruct((Bp, 16), jnp.float32),
        ],
        scratch_shapes=[pltpu.VMEM((Bp, 16), jnp.float32)],
    )(dbT3, db2, coord, vel, s0, s1, stats, vc9p,
      phivw1T, phivb1, phivw2T,
      rvw1T, rvb1, rvw2T, rvb2,
      vrw1T, vrb1, vrw2T, vrb2,
      cvw1r, cvb1, cvw2T, cvb2)


def kernel(edge_index, data_batch, coord, node_vel, virtual_coord, edge_attr,
           phi_w1, phi_b1, phi_w2, phiv_w1, phiv_b1, phiv_w2,
           em_w1, em_b1, em_w2, em_b2, rv_w1, rv_b1, rv_w2, rv_b2,
           vr_w1, vr_b1, vr_w2, vr_b2, cv_w1, cv_b1, cv_w2, cv_b2):
    f32 = jnp.float32
    row = edge_index[0].astype(jnp.int32)
    col = edge_index[1].astype(jnp.int32)
    coordp = jnp.zeros((N, 16), f32).at[:, :3].set(coord)

    ca, cb = _sc_gather(coordp, row, col)
    ca128 = ca.reshape(E // 8, 128)
    cb128 = cb.reshape(E // 8, 128)
    eaP = edge_attr.reshape(E // BLKE, EB8, 8, 4).transpose(0, 2, 1, 3).reshape(E, 4)
    rowP = row.reshape(E // BLKE, 2, EB16, 8).transpose(0, 2, 3, 1).reshape(E)
    trans128 = _edge_mlp(ca128, cb128, eaP, phi_w1, phi_b1, phi_w2,
                         em_w1, em_b1, em_w2, em_b2)
    trans = trans128.reshape(E, 8)
    zrows = jnp.zeros((ROWS_T, 8), f32)
    spart = _sc_scatter(trans, rowP, zrows)

    vc9p = jnp.zeros((Bp, 16), f32).at[:B, :9].set(virtual_coord.reshape(B, 9))
    dbi = data_batch.astype(jnp.int32)
    dbT3 = dbi.reshape(N // BLKD, 1, BLKD)
    db2 = dbi.reshape(N, 1)

    stats = _node_stats(dbT3, coord, vc9p)

    coord2, vout = _node_main(
        dbT3, db2, coord, node_vel, spart[0], spart[1], stats, vc9p,
        phiv_w1[:, :].T, phiv_b1.reshape(1, H), phiv_w2.T,
        rv_w1.T, rv_b1.reshape(1, H), rv_w2.T, rv_b2.reshape(1, 1),
        vr_w1.T, vr_b1.reshape(1, H), vr_w2.T, vr_b2.reshape(1, 1),
        cv_w1.reshape(1, H), cv_b1.reshape(1, H), cv_w2.T, cv_b2.reshape(1, 1))

    virtual_coord2 = vout[:B, :9].reshape(B, 3, C)
    return coord2, virtual_coord2


# speedup vs baseline: 1.2786x; 1.2786x over previous
import jax
import jax.numpy as jnp
from jax import lax
from jax.experimental import pallas as pl
from jax.experimental.pallas import tpu as pltpu
from jax.experimental.pallas import tpu_sc as plsc

N = 50000
E = 800000
B = 50
H = 64
C = 3
Bp = 64

NC, NS = 2, 16
NW = NC * NS
EW = E // NW
GK = 5000
ROWS_T = N // NS

BLKE = 6400
EB8 = BLKE // 8
EB16 = BLKE // 16
BLKD = 2000

_MESH_KW = dict(core_axis_name="c", subcore_axis_name="s",
                num_cores=NC, num_subcores=NS)


def _leaky(x):
    return jnp.where(x > 0, x, 0.2 * x)


def _dot(a, b):
    return jnp.dot(a, b, preferred_element_type=jnp.float32)


def _sc_gather(coordp, row, col):
    mesh = plsc.VectorSubcoreMesh(**_MESH_KW)

    def body(coordp_hbm, row_hbm, col_hbm, ca_hbm, cb_hbm, idx_v, rows_v, sem):
        wid = lax.axis_index("s") * NC + lax.axis_index("c")
        base = wid * EW
        for j in range(EW // GK):
            off = base + j * GK
            pltpu.sync_copy(row_hbm.at[pl.ds(off, GK)], idx_v)
            pltpu.async_copy(coordp_hbm.at[idx_v], rows_v, sem).wait()
            pltpu.sync_copy(rows_v, ca_hbm.at[pl.ds(off, GK)])
            pltpu.sync_copy(col_hbm.at[pl.ds(off, GK)], idx_v)
            pltpu.async_copy(coordp_hbm.at[idx_v], rows_v, sem).wait()
            pltpu.sync_copy(rows_v, cb_hbm.at[pl.ds(off, GK)])

    out_type = (jax.ShapeDtypeStruct((E, 16), jnp.float32),
                jax.ShapeDtypeStruct((E, 16), jnp.float32))
    return pl.kernel(
        body, out_type=out_type, mesh=mesh,
        compiler_params=pltpu.CompilerParams(use_tc_tiling_on_sc=False),
        scratch_types=[
            pltpu.VMEM((GK,), jnp.int32),
            pltpu.VMEM((GK, 16), jnp.float32),
            pltpu.SemaphoreType.DMA,
        ])(coordp, row, col)


def _edge_mlp(ca, cb, ea, phi_w1, phi_b1, phi_w2, em_w1, em_b1, em_w2, em_b2):
    w1r = phi_w1[:, 0].reshape(1, H)
    w1aT = phi_w1[:, 1:].T
    b1 = phi_b1.reshape(1, H)
    w2T = phi_w2.T
    emw1T = em_w1.T
    emb1 = em_b1.reshape(1, H)
    emw2T = em_w2.T
    emb2 = em_b2.reshape(1, 1)
    grid = E // BLKE

    def body(ca_ref, cb_ref, ea_ref, w1r_ref, w1aT_ref, b1_ref, w2T_ref,
             emw1T_ref, emb1_ref, emw2T_ref, emb2_ref, out_ref):
        d = ca_ref[...] - cb_ref[...]
        ea = ea_ref[...]
        pieces = []
        for k in range(8):
            dk = d[:, 16 * k:16 * k + 16]
            radial = jnp.sum(dk * dk, axis=1, keepdims=True)
            eak = ea[EB8 * k:EB8 * (k + 1), :]
            h1 = radial * w1r_ref[...] + _dot(eak, w1aT_ref[...]) + b1_ref[...]
            f = jnp.tanh(_dot(_leaky(h1), w2T_ref[...]))
            g = _leaky(_dot(f, emw1T_ref[...]) + emb1_ref[...])
            m = _dot(g, emw2T_ref[...]) + emb2_ref[...]
            cidx = lax.broadcasted_iota(jnp.int32, (EB8, 8), 1)
            tk = jnp.where(cidx < 3, dk[:, :8] * m,
                           jnp.where(cidx == 3, 1.0, 0.0))
            pieces.append(tk[:EB16])
            pieces.append(tk[EB16:])
        out_ref[...] = jnp.concatenate(pieces, axis=1)

    const = lambda shape: pl.BlockSpec(shape, lambda i: (0,) * len(shape))
    return pl.pallas_call(
        body,
        grid=(grid,),
        in_specs=[
            pl.BlockSpec((EB8, 128), lambda i: (i, 0)),
            pl.BlockSpec((EB8, 128), lambda i: (i, 0)),
            pl.BlockSpec((BLKE, 4), lambda i: (i, 0)),
            const((1, H)), const((4, H)), const((1, H)), const((H, H)),
            const((H, H)), const((1, H)), const((H, 1)), const((1, 1)),
        ],
        out_specs=pl.BlockSpec((EB16, 128), lambda i: (i, 0)),
        out_shape=jax.ShapeDtypeStruct((E // 16, 128), jnp.float32),
    )(ca, cb, ea, w1r, w1aT, b1, w2T, emw1T, emb1, emw2T, emb2)


def _sc_scatter(trans, row, zrows):
    mesh = plsc.VectorSubcoreMesh(**_MESH_KW)

    def body(trans_hbm, row_hbm, z_hbm, out_hbm, idx_v, tr_v, sem, acc):
        cid = lax.axis_index("c")
        sid = lax.axis_index("s")
        wid = sid * NC + cid
        pltpu.sync_copy(z_hbm, acc.at[pl.ds(sid * ROWS_T, ROWS_T)])
        plsc.subcore_barrier()
        base = wid * EW
        for j in range(EW // GK):
            off = base + j * GK
            pltpu.sync_copy(row_hbm.at[pl.ds(off, GK)], idx_v)
            pltpu.sync_copy(trans_hbm.at[pl.ds(off, GK)], tr_v)
            pltpu.sync_copy(tr_v, acc.at[idx_v], add=True)
        plsc.subcore_barrier()
        pltpu.sync_copy(acc.at[pl.ds(sid * ROWS_T, ROWS_T)],
                        out_hbm.at[cid, pl.ds(sid * ROWS_T, ROWS_T)])

    return pl.kernel(
        body,
        out_type=jax.ShapeDtypeStruct((NC, N, 8), jnp.float32),
        mesh=mesh,
        compiler_params=pltpu.CompilerParams(use_tc_tiling_on_sc=False),
        scratch_types=[
            pltpu.VMEM((GK,), jnp.int32),
            pltpu.VMEM((GK, 8), jnp.float32),
            pltpu.SemaphoreType.DMA,
            pltpu.VMEM_SHARED((N, 8), jnp.float32),
        ])(trans, row, zrows)


def _node_stats(dbT3, coord, vc9p):
    grid = N // BLKD

    def body(dbT_ref, coord_ref, vc9p_ref, out_ref, acc_ref):
        i = pl.program_id(0)

        @pl.when(i == 0)
        def _init():
            acc_ref[...] = jnp.zeros((Bp, 16), jnp.float32)

        dbt = dbT_ref[0]
        onehotT = (lax.broadcasted_iota(jnp.int32, (Bp, BLKD), 0)
                   == dbt).astype(jnp.float32)
        cx = jnp.concatenate(
            [coord_ref[...],
             jnp.ones((BLKD, 1), jnp.float32),
             jnp.zeros((BLKD, 12), jnp.float32)], axis=1)
        acc_ref[...] += _dot(onehotT, cx)

        @pl.when(i == grid - 1)
        def _fin():
            s = acc_ref[...]
            cnt = jnp.maximum(s[:, 3:4], 1.0)
            cm = s[:, 0:3] / cnt
            vc9 = vc9p_ref[...][:, 0:9]
            cm9 = jnp.concatenate(
                [cm[:, 0:1]] * 3 + [cm[:, 1:2]] * 3 + [cm[:, 2:3]] * 3, axis=1)
            A = vc9 - cm9
            cols = []
            for i_ in range(3):
                for j_ in range(3):
                    cols.append(A[:, i_:i_ + 1] * A[:, j_:j_ + 1]
                                + A[:, 3 + i_:4 + i_] * A[:, 3 + j_:4 + j_]
                                + A[:, 6 + i_:7 + i_] * A[:, 6 + j_:7 + j_])
            out_ref[...] = jnp.concatenate(
                [cm, cnt] + cols + [jnp.zeros((Bp, 3), jnp.float32)], axis=1)

    return pl.pallas_call(
        body,
        grid=(grid,),
        in_specs=[
            pl.BlockSpec((1, 1, BLKD), lambda i: (i, 0, 0)),
            pl.BlockSpec((BLKD, 3), lambda i: (i, 0)),
            pl.BlockSpec((Bp, 16), lambda i: (0, 0)),
        ],
        out_specs=pl.BlockSpec((Bp, 16), lambda i: (0, 0)),
        out_shape=jax.ShapeDtypeStruct((Bp, 16), jnp.float32),
        scratch_shapes=[pltpu.VMEM((Bp, 16), jnp.float32)],
    )(dbT3, coord, vc9p)


def _node_main(dbT3, db2, coord, vel, s0, s1, stats, vc9p,
               phivw1T, phivb1, phivw2T,
               rvw1T, rvb1, rvw2T, rvb2,
               vrw1T, vrb1, vrw2T, vrb2,
               cvw1r, cvb1, cvw2T, cvb2):
    grid = N // BLKD

    def body(dbT_ref, db_ref, coord_ref, vel_ref, s0_ref, s1_ref,
             stats_ref, vc9p_ref,
             phivw1T_ref, phivb1_ref, phivw2T_ref,
             rvw1T_ref, rvb1_ref, rvw2T_ref, rvb2_ref,
             vrw1T_ref, vrb1_ref, vrw2T_ref, vrb2_ref,
             cvw1r_ref, cvb1_ref, cvw2T_ref, cvb2_ref,
             coord2_ref, vout_ref, vacc_ref):
        i = pl.program_id(0)

        @pl.when(i == 0)
        def _init():
            vacc_ref[...] = jnp.zeros((Bp, 16), jnp.float32)

        st = stats_ref[...]
        cnt_g = st[:, 3:4]
        mX9 = st[:, 4:13]
        vc9 = vc9p_ref[...][:, 0:9]

        db = db_ref[...]
        onehot = (db == lax.broadcasted_iota(jnp.int32, (BLKD, Bp), 1)
                  ).astype(jnp.float32)
        dbt = dbT_ref[0]
        onehotT = (lax.broadcasted_iota(jnp.int32, (Bp, BLKD), 0)
                   == dbt).astype(jnp.float32)

        co = coord_ref[...]
        vcdb = _dot(onehot, vc9)
        mXdb = _dot(onehot, mX9)
        co9 = jnp.concatenate(
            [co[:, 0:1]] * 3 + [co[:, 1:2]] * 3 + [co[:, 2:3]] * 3, axis=1)
        vcd9 = vcdb - co9
        sq = vcd9 * vcd9
        vrad = jnp.sqrt(sq[:, 0:3] + sq[:, 3:6] + sq[:, 6:9])

        rv_cols, vr_cols = [], []
        for c in range(C):
            Xc = jnp.concatenate(
                [vrad[:, c:c + 1], mXdb[:, c:c + 1],
                 mXdb[:, 3 + c:4 + c], mXdb[:, 6 + c:7 + c]], axis=1)
            h = _leaky(_dot(Xc, phivw1T_ref[...]) + phivb1_ref[...])
            vef = jnp.tanh(_dot(h, phivw2T_ref[...]))
            hr = _leaky(_dot(vef, rvw1T_ref[...]) + rvb1_ref[...])
            rv_cols.append(_dot(hr, rvw2T_ref[...]) + rvb2_ref[...])
            hv = _leaky(_dot(vef, vrw1T_ref[...]) + vrb1_ref[...])
            vr_cols.append(_dot(hv, vrw2T_ref[...]) + vrb2_ref[...])
        rv = jnp.concatenate(rv_cols, axis=1)
        vr = jnp.concatenate(vr_cols, axis=1)

        rv9 = jnp.concatenate([rv] * 3, axis=1)
        prod = vcd9 * rv9
        vterm = -jnp.concatenate(
            [jnp.sum(prod[:, 0:3], axis=1, keepdims=True),
             jnp.sum(prod[:, 3:6], axis=1, keepdims=True),
             jnp.sum(prod[:, 6:9], axis=1, keepdims=True)], axis=1) / 3.0

        vr9 = jnp.concatenate([vr] * 3, axis=1)
        trans2 = vcd9 * vr9

        v = vel_ref[...]
        vnorm = jnp.sqrt(jnp.sum(v * v, axis=1, keepdims=True))
        hc = _leaky(vnorm * cvw1r_ref[...] + cvb1_ref[...])
        cvout = _dot(hc, cvw2T_ref[...]) + cvb2_ref[...]

        es = s0_ref[...] + s1_ref[...]
        emean = es[:, 0:3] / jnp.maximum(es[:, 3:4], 1.0)

        coord2_ref[...] = co + emean + vterm + v * cvout

        tr2p = jnp.concatenate(
            [trans2, jnp.zeros((BLKD, 7), jnp.float32)], axis=1)
        vacc_ref[...] += _dot(onehotT, tr2p)

        @pl.when(i == grid - 1)
        def _fin():
            vout_ref[...] = vc9p_ref[...] + vacc_ref[...] / cnt_g

    const = lambda shape: pl.BlockSpec(shape, lambda i: (0,) * len(shape))
    return pl.pallas_call(
        body,
        grid=(grid,),
        in_specs=[
            pl.BlockSpec((1, 1, BLKD), lambda i: (i, 0, 0)),
            pl.BlockSpec((BLKD, 1), lambda i: (i, 0)),
            pl.BlockSpec((BLKD, 3), lambda i: (i, 0)),
            pl.BlockSpec((BLKD, 3), lambda i: (i, 0)),
            pl.BlockSpec((BLKD, 8), lambda i: (i, 0)),
            pl.BlockSpec((BLKD, 8), lambda i: (i, 0)),
            const((Bp, 16)), const((Bp, 16)),
            const((4, H)), const((1, H)), const((H, H)),
            const((H, H)), const((1, H)), const((H, 1)), const((1, 1)),
            const((H, H)), const((1, H)), const((H, 1)), const((1, 1)),
            const((1, H)), const((1, H)), const((H, 1)), const((1, 1)),
        ],
        out_specs=[
            pl.BlockSpec((BLKD, 3), lambda i: (i, 0)),
            pl.BlockSpec((Bp, 16), lambda i: (0, 0)),
        ],
        out_shape=[
            jax.ShapeDtypeStruct((N, 3), jnp.float32),
            jax.ShapeDtypeStruct((Bp, 16), jnp.float32),
        ],
        scratch_shapes=[pltpu.VMEM((Bp, 16), jnp.float32)],
    )(dbT3, db2, coord, vel, s0, s1, stats, vc9p,
      phivw1T, phivb1, phivw2T,
      rvw1T, rvb1, rvw2T, rvb2,
      vrw1T, vrb1, vrw2T, vrb2,
      cvw1r, cvb1, cvw2T, cvb2)


def kernel(edge_index, data_batch, coord, node_vel, virtual_coord, edge_attr,
           phi_w1, phi_b1, phi_w2, phiv_w1, phiv_b1, phiv_w2,
           em_w1, em_b1, em_w2, em_b2, rv_w1, rv_b1, rv_w2, rv_b2,
           vr_w1, vr_b1, vr_w2, vr_b2, cv_w1, cv_b1, cv_w2, cv_b2):
    f32 = jnp.float32
    row = edge_index[0].astype(jnp.int32)
    col = edge_index[1].astype(jnp.int32)
    coordp = jnp.zeros((N, 16), f32).at[:, :3].set(coord)

    rowS = row.reshape(E // BLKE, 8, EB8).transpose(0, 2, 1).reshape(E)
    colS = col.reshape(E // BLKE, 8, EB8).transpose(0, 2, 1).reshape(E)
    ca, cb = _sc_gather(coordp, rowS, colS)
    ca128 = ca.reshape(E // 8, 128)
    cb128 = cb.reshape(E // 8, 128)
    trans128 = _edge_mlp(ca128, cb128, edge_attr, phi_w1, phi_b1, phi_w2,
                         em_w1, em_b1, em_w2, em_b2)
    trans = trans128.reshape(E, 8)
    rowP = row.reshape(E // BLKE, 8, 2, EB16).transpose(0, 3, 1, 2).reshape(E)
    zrows = jnp.zeros((ROWS_T, 8), f32)
    spart = _sc_scatter(trans, rowP, zrows)

    vc9p = jnp.zeros((Bp, 16), f32).at[:B, :9].set(virtual_coord.reshape(B, 9))
    dbi = data_batch.astype(jnp.int32)
    dbT3 = dbi.reshape(N // BLKD, 1, BLKD)
    db2 = dbi.reshape(N, 1)

    stats = _node_stats(dbT3, coord, vc9p)

    coord2, vout = _node_main(
        dbT3, db2, coord, node_vel, spart[0], spart[1], stats, vc9p,
        phiv_w1[:, :].T, phiv_b1.reshape(1, H), phiv_w2.T,
        rv_w1.T, rv_b1.reshape(1, H), rv_w2.T, rv_b2.reshape(1, 1),
        vr_w1.T, vr_b1.reshape(1, H), vr_w2.T, vr_b2.reshape(1, 1),
        cv_w1.reshape(1, H), cv_b1.reshape(1, H), cv_w2.T, cv_b2.reshape(1, 1))

    virtual_coord2 = vout[:B, :9].reshape(B, 3, C)
    return coord2, virtual_coord2


# speedup vs baseline: 1.4826x; 1.1595x over previous
import jax
import jax.numpy as jnp
from jax import lax
from jax.experimental import pallas as pl
from jax.experimental.pallas import tpu as pltpu
from jax.experimental.pallas import tpu_sc as plsc

N = 50000
E = 800000
B = 50
H = 64
C = 3
Bp = 64

NC, NS = 2, 16
NW = NC * NS
EW = E // NW
GK = 5000
ROWS_T = N // NS

BLKE = 6400
EB8 = BLKE // 8
EB16 = BLKE // 16
BLKD = 2000

_MESH_KW = dict(core_axis_name="c", subcore_axis_name="s",
                num_cores=NC, num_subcores=NS)


def _leaky(x):
    return jnp.where(x > 0, x, 0.2 * x)


def _dot(a, b):
    return jnp.dot(a, b, preferred_element_type=jnp.float32)


def _sc_gather(coordp, row, col):
    mesh = plsc.VectorSubcoreMesh(**_MESH_KW)

    def body(coordp_hbm, row_hbm, col_hbm, ca_hbm, cb_hbm, idx_v, rows_v, sem):
        wid = lax.axis_index("s") * NC + lax.axis_index("c")
        base = wid * EW
        for j in range(EW // GK):
            off = base + j * GK
            pltpu.sync_copy(row_hbm.at[pl.ds(off, GK)], idx_v)
            pltpu.async_copy(coordp_hbm.at[idx_v], rows_v, sem).wait()
            pltpu.sync_copy(rows_v, ca_hbm.at[pl.ds(off, GK)])
            pltpu.sync_copy(col_hbm.at[pl.ds(off, GK)], idx_v)
            pltpu.async_copy(coordp_hbm.at[idx_v], rows_v, sem).wait()
            pltpu.sync_copy(rows_v, cb_hbm.at[pl.ds(off, GK)])

    out_type = (jax.ShapeDtypeStruct((E, 16), jnp.float32),
                jax.ShapeDtypeStruct((E, 16), jnp.float32))
    return pl.kernel(
        body, out_type=out_type, mesh=mesh,
        compiler_params=pltpu.CompilerParams(use_tc_tiling_on_sc=False),
        scratch_types=[
            pltpu.VMEM((GK,), jnp.int32),
            pltpu.VMEM((GK, 16), jnp.float32),
            pltpu.SemaphoreType.DMA,
        ])(coordp, row, col)


def _edge_mlp(ca, cb, ea, phi_w1, phi_b1, phi_w2, em_w1, em_b1, em_w2, em_b2):
    w15 = phi_w1.T
    b1 = phi_b1.reshape(1, H)
    lane = jnp.arange(128)[:, None]
    sel = ((lane // 16 == jnp.arange(8)[None, :]) & (lane % 16 < 3)
           ).astype(jnp.float32)
    w2T = phi_w2.T
    emw1T = em_w1.T
    emb1 = em_b1.reshape(1, H)
    emw2T8 = jnp.tile(em_w2.T, (1, 8))
    emb2 = jnp.tile(em_b2.reshape(1, 1), (1, 8))
    grid = E // BLKE

    def body(ca_ref, cb_ref, ea_ref, sel_ref, w15_ref, b1_ref, w2T_ref,
             emw1T_ref, emb1_ref, emw2T8_ref, emb2_ref, out_ref):
        d = ca_ref[...] - cb_ref[...]
        radp = _dot(d * d, sel_ref[...])
        rad = jnp.concatenate(
            [radp[:, k:k + 1] for k in range(8)], axis=0)
        x5 = jnp.concatenate([rad, ea_ref[...]], axis=1)
        h1 = _dot(x5, w15_ref[...]) + b1_ref[...]
        f = jnp.tanh(_dot(_leaky(h1), w2T_ref[...]))
        g = _leaky(_dot(f, emw1T_ref[...]) + emb1_ref[...])
        m8 = _dot(g, emw2T8_ref[...]) + emb2_ref[...]
        cone = (lax.broadcasted_iota(jnp.int32, (1, 8), 1) == 3
                ).astype(jnp.float32)
        pieces = []
        for k in range(8):
            dk = d[:, 16 * k:16 * k + 8]
            tk = dk * m8[EB8 * k:EB8 * (k + 1), :] + cone
            pieces.append(tk[:EB16])
            pieces.append(tk[EB16:])
        out_ref[...] = jnp.concatenate(pieces, axis=1)

    const = lambda shape: pl.BlockSpec(shape, lambda i: (0,) * len(shape))
    return pl.pallas_call(
        body,
        grid=(grid,),
        in_specs=[
            pl.BlockSpec((EB8, 128), lambda i: (i, 0)),
            pl.BlockSpec((EB8, 128), lambda i: (i, 0)),
            pl.BlockSpec((BLKE, 4), lambda i: (i, 0)),
            const((128, 8)), const((5, H)), const((1, H)), const((H, H)),
            const((H, H)), const((1, H)), const((H, 8)), const((1, 8)),
        ],
        out_specs=pl.BlockSpec((EB16, 128), lambda i: (i, 0)),
        out_shape=jax.ShapeDtypeStruct((E // 16, 128), jnp.float32),
    )(ca, cb, ea, sel, w15, b1, w2T, emw1T, emb1, emw2T8, emb2)


def _sc_scatter(trans, row, zrows):
    mesh = plsc.VectorSubcoreMesh(**_MESH_KW)

    def body(trans_hbm, row_hbm, z_hbm, out_hbm, idx_v, tr_v, sem, acc):
        cid = lax.axis_index("c")
        sid = lax.axis_index("s")
        wid = sid * NC + cid
        pltpu.sync_copy(z_hbm, acc.at[pl.ds(sid * ROWS_T, ROWS_T)])
        plsc.subcore_barrier()
        base = wid * EW
        for j in range(EW // GK):
            off = base + j * GK
            pltpu.sync_copy(row_hbm.at[pl.ds(off, GK)], idx_v)
            pltpu.sync_copy(trans_hbm.at[pl.ds(off, GK)], tr_v)
            pltpu.sync_copy(tr_v, acc.at[idx_v], add=True)
        plsc.subcore_barrier()
        pltpu.sync_copy(acc.at[pl.ds(sid * ROWS_T, ROWS_T)],
                        out_hbm.at[cid, pl.ds(sid * ROWS_T, ROWS_T)])

    return pl.kernel(
        body,
        out_type=jax.ShapeDtypeStruct((NC, N, 8), jnp.float32),
        mesh=mesh,
        compiler_params=pltpu.CompilerParams(use_tc_tiling_on_sc=False),
        scratch_types=[
            pltpu.VMEM((GK,), jnp.int32),
            pltpu.VMEM((GK, 8), jnp.float32),
            pltpu.SemaphoreType.DMA,
            pltpu.VMEM_SHARED((N, 8), jnp.float32),
        ])(trans, row, zrows)


def _node_stats(dbT3, coord, vc9p):
    grid = N // BLKD

    def body(dbT_ref, coord_ref, vc9p_ref, out_ref, acc_ref):
        i = pl.program_id(0)

        @pl.when(i == 0)
        def _init():
            acc_ref[...] = jnp.zeros((Bp, 16), jnp.float32)

        dbt = dbT_ref[0]
        onehotT = (lax.broadcasted_iota(jnp.int32, (Bp, BLKD), 0)
                   == dbt).astype(jnp.float32)
        cx = jnp.concatenate(
            [coord_ref[...],
             jnp.ones((BLKD, 1), jnp.float32),
             jnp.zeros((BLKD, 12), jnp.float32)], axis=1)
        acc_ref[...] += _dot(onehotT, cx)

        @pl.when(i == grid - 1)
        def _fin():
            s = acc_ref[...]
            cnt = jnp.maximum(s[:, 3:4], 1.0)
            cm = s[:, 0:3] / cnt
            vc9 = vc9p_ref[...][:, 0:9]
            cm9 = jnp.concatenate(
                [cm[:, 0:1]] * 3 + [cm[:, 1:2]] * 3 + [cm[:, 2:3]] * 3, axis=1)
            A = vc9 - cm9
            cols = []
            for i_ in range(3):
                for j_ in range(3):
                    cols.append(A[:, i_:i_ + 1] * A[:, j_:j_ + 1]
                                + A[:, 3 + i_:4 + i_] * A[:, 3 + j_:4 + j_]
                                + A[:, 6 + i_:7 + i_] * A[:, 6 + j_:7 + j_])
            out_ref[...] = jnp.concatenate(
                [cm, cnt] + cols + [jnp.zeros((Bp, 3), jnp.float32)], axis=1)

    return pl.pallas_call(
        body,
        grid=(grid,),
        in_specs=[
            pl.BlockSpec((1, 1, BLKD), lambda i: (i, 0, 0)),
            pl.BlockSpec((BLKD, 3), lambda i: (i, 0)),
            pl.BlockSpec((Bp, 16), lambda i: (0, 0)),
        ],
        out_specs=pl.BlockSpec((Bp, 16), lambda i: (0, 0)),
        out_shape=jax.ShapeDtypeStruct((Bp, 16), jnp.float32),
        scratch_shapes=[pltpu.VMEM((Bp, 16), jnp.float32)],
    )(dbT3, coord, vc9p)


def _node_main(dbT3, db2, coord, vel, s0, s1, stats, vc9p,
               phivw1T, phivb1, phivw2T,
               rvw1T, rvb1, rvw2T, rvb2,
               vrw1T, vrb1, vrw2T, vrb2,
               cvw1r, cvb1, cvw2T, cvb2):
    grid = N // BLKD

    def body(dbT_ref, db_ref, coord_ref, vel_ref, s0_ref, s1_ref,
             stats_ref, vc9p_ref,
             phivw1T_ref, phivb1_ref, phivw2T_ref,
             rvw1T_ref, rvb1_ref, rvw2T_ref, rvb2_ref,
             vrw1T_ref, vrb1_ref, vrw2T_ref, vrb2_ref,
             cvw1r_ref, cvb1_ref, cvw2T_ref, cvb2_ref,
             coord2_ref, vout_ref, vacc_ref):
        i = pl.program_id(0)

        @pl.when(i == 0)
        def _init():
            vacc_ref[...] = jnp.zeros((Bp, 16), jnp.float32)

        st = stats_ref[...]
        cnt_g = st[:, 3:4]
        mX9 = st[:, 4:13]
        vc9 = vc9p_ref[...][:, 0:9]

        db = db_ref[...]
        onehot = (db == lax.broadcasted_iota(jnp.int32, (BLKD, Bp), 1)
                  ).astype(jnp.float32)
        dbt = dbT_ref[0]
        onehotT = (lax.broadcasted_iota(jnp.int32, (Bp, BLKD), 0)
                   == dbt).astype(jnp.float32)

        co = coord_ref[...]
        gathered = _dot(onehot, jnp.concatenate([vc9, mX9], axis=1))
        vcdb = gathered[:, 0:9]
        mXdb = gathered[:, 9:18]
        co9 = jnp.concatenate(
            [co[:, 0:1]] * 3 + [co[:, 1:2]] * 3 + [co[:, 2:3]] * 3, axis=1)
        vcd9 = vcdb - co9
        sq = vcd9 * vcd9
        vrad = jnp.sqrt(sq[:, 0:3] + sq[:, 3:6] + sq[:, 6:9])

        X = jnp.concatenate(
            [jnp.concatenate(
                [vrad[:, c:c + 1], mXdb[:, c:c + 1],
                 mXdb[:, 3 + c:4 + c], mXdb[:, 6 + c:7 + c]], axis=1)
             for c in range(C)], axis=0)
        h = _leaky(_dot(X, phivw1T_ref[...]) + phivb1_ref[...])
        vef = jnp.tanh(_dot(h, phivw2T_ref[...]))
        hr = _leaky(_dot(vef, rvw1T_ref[...]) + rvb1_ref[...])
        rva = _dot(hr, rvw2T_ref[...]) + rvb2_ref[...]
        hv = _leaky(_dot(vef, vrw1T_ref[...]) + vrb1_ref[...])
        vra = _dot(hv, vrw2T_ref[...]) + vrb2_ref[...]
        rv = jnp.concatenate(
            [rva[BLKD * c:BLKD * (c + 1), :] for c in range(C)], axis=1)
        vr = jnp.concatenate(
            [vra[BLKD * c:BLKD * (c + 1), :] for c in range(C)], axis=1)

        rv9 = jnp.concatenate([rv] * 3, axis=1)
        prod = vcd9 * rv9
        vterm = -jnp.concatenate(
            [jnp.sum(prod[:, 0:3], axis=1, keepdims=True),
             jnp.sum(prod[:, 3:6], axis=1, keepdims=True),
             jnp.sum(prod[:, 6:9], axis=1, keepdims=True)], axis=1) / 3.0

        vr9 = jnp.concatenate([vr] * 3, axis=1)
        trans2 = vcd9 * vr9

        v = vel_ref[...]
        vnorm = jnp.sqrt(jnp.sum(v * v, axis=1, keepdims=True))
        hc = _leaky(vnorm * cvw1r_ref[...] + cvb1_ref[...])
        cvout = _dot(hc, cvw2T_ref[...]) + cvb2_ref[...]

        es = s0_ref[...] + s1_ref[...]
        emean = es[:, 0:3] / jnp.maximum(es[:, 3:4], 1.0)

        coord2_ref[...] = co + emean + vterm + v * cvout

        tr2p = jnp.concatenate(
            [trans2, jnp.zeros((BLKD, 7), jnp.float32)], axis=1)
        vacc_ref[...] += _dot(onehotT, tr2p)

        @pl.when(i == grid - 1)
        def _fin():
            vout_ref[...] = vc9p_ref[...] + vacc_ref[...] / cnt_g

    const = lambda shape: pl.BlockSpec(shape, lambda i: (0,) * len(shape))
    return pl.pallas_call(
        body,
        grid=(grid,),
        in_specs=[
            pl.BlockSpec((1, 1, BLKD), lambda i: (i, 0, 0)),
            pl.BlockSpec((BLKD, 1), lambda i: (i, 0)),
            pl.BlockSpec((BLKD, 3), lambda i: (i, 0)),
            pl.BlockSpec((BLKD, 3), lambda i: (i, 0)),
            pl.BlockSpec((BLKD, 8), lambda i: (i, 0)),
            pl.BlockSpec((BLKD, 8), lambda i: (i, 0)),
            const((Bp, 16)), const((Bp, 16)),
            const((4, H)), const((1, H)), const((H, H)),
            const((H, H)), const((1, H)), const((H, 1)), const((1, 1)),
            const((H, H)), const((1, H)), const((H, 1)), const((1, 1)),
            const((1, H)), const((1, H)), const((H, 1)), const((1, 1)),
        ],
        out_specs=[
            pl.BlockSpec((BLKD, 3), lambda i: (i, 0)),
            pl.BlockSpec((Bp, 16), lambda i: (0, 0)),
        ],
        out_shape=[
            jax.ShapeDtypeStruct((N, 3), jnp.float32),
            jax.ShapeDtypeStruct((Bp, 16), jnp.float32),
        ],
        scratch_shapes=[pltpu.VMEM((Bp, 16), jnp.float32)],
    )(dbT3, db2, coord, vel, s0, s1, stats, vc9p,
      phivw1T, phivb1, phivw2T,
      rvw1T, rvb1, rvw2T, rvb2,
      vrw1T, vrb1, vrw2T, vrb2,
      cvw1r, cvb1, cvw2T, cvb2)


def kernel(edge_index, data_batch, coord, node_vel, virtual_coord, edge_attr,
           phi_w1, phi_b1, phi_w2, phiv_w1, phiv_b1, phiv_w2,
           em_w1, em_b1, em_w2, em_b2, rv_w1, rv_b1, rv_w2, rv_b2,
           vr_w1, vr_b1, vr_w2, vr_b2, cv_w1, cv_b1, cv_w2, cv_b2):
    f32 = jnp.float32
    row = edge_index[0].astype(jnp.int32)
    col = edge_index[1].astype(jnp.int32)
    coordp = jnp.zeros((N, 16), f32).at[:, :3].set(coord)

    rowS = row.reshape(E // BLKE, 8, EB8).transpose(0, 2, 1).reshape(E)
    colS = col.reshape(E // BLKE, 8, EB8).transpose(0, 2, 1).reshape(E)
    ca, cb = _sc_gather(coordp, rowS, colS)
    ca128 = ca.reshape(E // 8, 128)
    cb128 = cb.reshape(E // 8, 128)
    trans128 = _edge_mlp(ca128, cb128, edge_attr, phi_w1, phi_b1, phi_w2,
                         em_w1, em_b1, em_w2, em_b2)
    trans = trans128.reshape(E, 8)
    rowP = row.reshape(E // BLKE, 8, 2, EB16).transpose(0, 3, 1, 2).reshape(E)
    zrows = jnp.zeros((ROWS_T, 8), f32)
    spart = _sc_scatter(trans, rowP, zrows)

    vc9p = jnp.zeros((Bp, 16), f32).at[:B, :9].set(virtual_coord.reshape(B, 9))
    dbi = data_batch.astype(jnp.int32)
    dbT3 = dbi.reshape(N // BLKD, 1, BLKD)
    db2 = dbi.reshape(N, 1)

    stats = _node_stats(dbT3, coord, vc9p)

    coord2, vout = _node_main(
        dbT3, db2, coord, node_vel, spart[0], spart[1], stats, vc9p,
        phiv_w1[:, :].T, phiv_b1.reshape(1, H), phiv_w2.T,
        rv_w1.T, rv_b1.reshape(1, H), rv_w2.T, rv_b2.reshape(1, 1),
        vr_w1.T, vr_b1.reshape(1, H), vr_w2.T, vr_b2.reshape(1, 1),
        cv_w1.reshape(1, H), cv_b1.reshape(1, H), cv_w2.T, cv_b2.reshape(1, 1))

    virtual_coord2 = vout[:B, :9].reshape(B, 3, C)
    return coord2, virtual_coord2


# speedup vs baseline: 1.9329x; 1.3037x over previous
import jax
import jax.numpy as jnp
from jax import lax
from jax.experimental import pallas as pl
from jax.experimental.pallas import tpu as pltpu
from jax.experimental.pallas import tpu_sc as plsc

N = 50000
E = 800000
B = 50
H = 64
C = 3
Bp = 64

NC, NS = 2, 16
NW = NC * NS
EW = E // NW
GK = 5000
ROWS_T = N // NS

BLKE = 6400
EB8 = BLKE // 8
EB16 = BLKE // 16
BLKD = 2000

_MESH_KW = dict(core_axis_name="c", subcore_axis_name="s",
                num_cores=NC, num_subcores=NS)


def _leaky(x):
    return jnp.where(x > 0, x, 0.2 * x)


def _dot(a, b):
    return jnp.dot(a, b, preferred_element_type=jnp.float32)


def _sc_gather(coordp, row, col):
    mesh = plsc.VectorSubcoreMesh(**_MESH_KW)

    def body(coordp_hbm, row_hbm, col_hbm, ca_hbm, cb_hbm, idx_v, rows_v, sem):
        wid = lax.axis_index("s") * NC + lax.axis_index("c")
        base = wid * EW
        for j in range(EW // GK):
            off = base + j * GK
            pltpu.sync_copy(row_hbm.at[pl.ds(off, GK)], idx_v)
            pltpu.async_copy(coordp_hbm.at[idx_v], rows_v, sem).wait()
            pltpu.sync_copy(rows_v, ca_hbm.at[pl.ds(off, GK)])
            pltpu.sync_copy(col_hbm.at[pl.ds(off, GK)], idx_v)
            pltpu.async_copy(coordp_hbm.at[idx_v], rows_v, sem).wait()
            pltpu.sync_copy(rows_v, cb_hbm.at[pl.ds(off, GK)])

    out_type = (jax.ShapeDtypeStruct((E, 8), jnp.float32),
                jax.ShapeDtypeStruct((E, 8), jnp.float32))
    return pl.kernel(
        body, out_type=out_type, mesh=mesh,
        compiler_params=pltpu.CompilerParams(use_tc_tiling_on_sc=False),
        scratch_types=[
            pltpu.VMEM((GK,), jnp.int32),
            pltpu.VMEM((GK, 8), jnp.float32),
            pltpu.SemaphoreType.DMA,
        ])(coordp, row, col)


def _edge_mlp(ca, cb, ea, phi_w1, phi_b1, phi_w2, em_w1, em_b1, em_w2, em_b2):
    w15 = phi_w1.T
    b1 = phi_b1.reshape(1, H)
    lane = jnp.arange(128)[:, None]
    sel = ((lane // 8 == jnp.arange(16)[None, :]) & (lane % 8 < 3)
           ).astype(jnp.float32)
    w2T = phi_w2.T
    emw1T = em_w1.T
    emb1 = em_b1.reshape(1, H)
    emw2T8 = jnp.tile(em_w2.T, (1, 16))
    emb2 = jnp.tile(em_b2.reshape(1, 1), (1, 16))
    grid = E // BLKE

    def body(ca_ref, cb_ref, ea_ref, sel_ref, w15_ref, b1_ref, w2T_ref,
             emw1T_ref, emb1_ref, emw2T8_ref, emb2_ref, out_ref):
        d = ca_ref[...] - cb_ref[...]
        radp = _dot(d * d, sel_ref[...])
        rad = jnp.concatenate(
            [radp[:, k:k + 1] for k in range(16)], axis=0)
        ea = ea_ref[...]
        eas = jnp.concatenate(
            [ea[:, 8 * k:8 * k + 4] for k in range(16)], axis=0)
        x5 = jnp.concatenate([rad, eas], axis=1)
        h1 = _dot(x5, w15_ref[...]) + b1_ref[...]
        f = jnp.tanh(_dot(_leaky(h1), w2T_ref[...]))
        g = _leaky(_dot(f, emw1T_ref[...]) + emb1_ref[...])
        m16 = _dot(g, emw2T8_ref[...]) + emb2_ref[...]
        cone = (lax.broadcasted_iota(jnp.int32, (1, 8), 1) == 3
                ).astype(jnp.float32)
        pieces = []
        for k in range(16):
            dk = d[:, 8 * k:8 * k + 8]
            tk = dk * m16[EB16 * k:EB16 * (k + 1), 0:8] + cone
            pieces.append(tk)
        out_ref[...] = jnp.concatenate(pieces, axis=1)

    const = lambda shape: pl.BlockSpec(shape, lambda i: (0,) * len(shape))
    return pl.pallas_call(
        body,
        grid=(grid,),
        in_specs=[
            pl.BlockSpec((EB16, 128), lambda i: (i, 0)),
            pl.BlockSpec((EB16, 128), lambda i: (i, 0)),
            pl.BlockSpec((EB16, 128), lambda i: (i, 0)),
            const((128, 16)), const((5, H)), const((1, H)), const((H, H)),
            const((H, H)), const((1, H)), const((H, 16)), const((1, 16)),
        ],
        out_specs=pl.BlockSpec((EB16, 128), lambda i: (i, 0)),
        out_shape=jax.ShapeDtypeStruct((E // 16, 128), jnp.float32),
    )(ca, cb, ea, sel, w15, b1, w2T, emw1T, emb1, emw2T8, emb2)


def _sc_scatter(trans, row, zrows):
    mesh = plsc.VectorSubcoreMesh(**_MESH_KW)

    def body(trans_hbm, row_hbm, z_hbm, out_hbm, idx_v, tr_v, sem, acc):
        cid = lax.axis_index("c")
        sid = lax.axis_index("s")
        wid = sid * NC + cid
        pltpu.sync_copy(z_hbm, acc.at[pl.ds(sid * ROWS_T, ROWS_T)])
        plsc.subcore_barrier()
        base = wid * EW
        for j in range(EW // GK):
            off = base + j * GK
            pltpu.sync_copy(row_hbm.at[pl.ds(off, GK)], idx_v)
            pltpu.sync_copy(trans_hbm.at[pl.ds(off, GK)], tr_v)
            pltpu.sync_copy(tr_v, acc.at[idx_v], add=True)
        plsc.subcore_barrier()
        pltpu.sync_copy(acc.at[pl.ds(sid * ROWS_T, ROWS_T)],
                        out_hbm.at[cid, pl.ds(sid * ROWS_T, ROWS_T)])

    return pl.kernel(
        body,
        out_type=jax.ShapeDtypeStruct((NC, N, 8), jnp.float32),
        mesh=mesh,
        compiler_params=pltpu.CompilerParams(use_tc_tiling_on_sc=False),
        scratch_types=[
            pltpu.VMEM((GK,), jnp.int32),
            pltpu.VMEM((GK, 8), jnp.float32),
            pltpu.SemaphoreType.DMA,
            pltpu.VMEM_SHARED((N, 8), jnp.float32),
        ])(trans, row, zrows)


def _node_stats(dbT3, coord, vc9p):
    grid = N // BLKD

    def body(dbT_ref, coord_ref, vc9p_ref, out_ref, acc_ref):
        i = pl.program_id(0)

        @pl.when(i == 0)
        def _init():
            acc_ref[...] = jnp.zeros((Bp, 16), jnp.float32)

        dbt = dbT_ref[0]
        onehotT = (lax.broadcasted_iota(jnp.int32, (Bp, BLKD), 0)
                   == dbt).astype(jnp.float32)
        cx = jnp.concatenate(
            [coord_ref[...],
             jnp.ones((BLKD, 1), jnp.float32),
             jnp.zeros((BLKD, 12), jnp.float32)], axis=1)
        acc_ref[...] += _dot(onehotT, cx)

        @pl.when(i == grid - 1)
        def _fin():
            s = acc_ref[...]
            cnt = jnp.maximum(s[:, 3:4], 1.0)
            cm = s[:, 0:3] / cnt
            vc9 = vc9p_ref[...][:, 0:9]
            cm9 = jnp.concatenate(
                [cm[:, 0:1]] * 3 + [cm[:, 1:2]] * 3 + [cm[:, 2:3]] * 3, axis=1)
            A = vc9 - cm9
            cols = []
            for i_ in range(3):
                for j_ in range(3):
                    cols.append(A[:, i_:i_ + 1] * A[:, j_:j_ + 1]
                                + A[:, 3 + i_:4 + i_] * A[:, 3 + j_:4 + j_]
                                + A[:, 6 + i_:7 + i_] * A[:, 6 + j_:7 + j_])
            out_ref[...] = jnp.concatenate(
                [cm, cnt] + cols + [jnp.zeros((Bp, 3), jnp.float32)], axis=1)

    return pl.pallas_call(
        body,
        grid=(grid,),
        in_specs=[
            pl.BlockSpec((1, 1, BLKD), lambda i: (i, 0, 0)),
            pl.BlockSpec((BLKD, 3), lambda i: (i, 0)),
            pl.BlockSpec((Bp, 16), lambda i: (0, 0)),
        ],
        out_specs=pl.BlockSpec((Bp, 16), lambda i: (0, 0)),
        out_shape=jax.ShapeDtypeStruct((Bp, 16), jnp.float32),
        scratch_shapes=[pltpu.VMEM((Bp, 16), jnp.float32)],
    )(dbT3, coord, vc9p)


def _node_main(dbT3, db2, coord, vel, s0, s1, stats, vc9p,
               phivw1T, phivb1, phivw2T,
               rvw1T, rvb1, rvw2T, rvb2,
               vrw1T, vrb1, vrw2T, vrb2,
               cvw1r, cvb1, cvw2T, cvb2):
    grid = N // BLKD

    def body(dbT_ref, db_ref, coord_ref, vel_ref, s0_ref, s1_ref,
             stats_ref, vc9p_ref,
             phivw1T_ref, phivb1_ref, phivw2T_ref,
             rvw1T_ref, rvb1_ref, rvw2T_ref, rvb2_ref,
             vrw1T_ref, vrb1_ref, vrw2T_ref, vrb2_ref,
             cvw1r_ref, cvb1_ref, cvw2T_ref, cvb2_ref,
             coord2_ref, vout_ref, vacc_ref):
        i = pl.program_id(0)

        @pl.when(i == 0)
        def _init():
            vacc_ref[...] = jnp.zeros((Bp, 16), jnp.float32)

        st = stats_ref[...]
        cnt_g = st[:, 3:4]
        mX9 = st[:, 4:13]
        vc9 = vc9p_ref[...][:, 0:9]

        db = db_ref[...]
        onehot = (db == lax.broadcasted_iota(jnp.int32, (BLKD, Bp), 1)
                  ).astype(jnp.float32)
        dbt = dbT_ref[0]
        onehotT = (lax.broadcasted_iota(jnp.int32, (Bp, BLKD), 0)
                   == dbt).astype(jnp.float32)

        co = coord_ref[...]
        gathered = _dot(onehot, jnp.concatenate([vc9, mX9], axis=1))
        vcdb = gathered[:, 0:9]
        mXdb = gathered[:, 9:18]
        co9 = jnp.concatenate(
            [co[:, 0:1]] * 3 + [co[:, 1:2]] * 3 + [co[:, 2:3]] * 3, axis=1)
        vcd9 = vcdb - co9
        sq = vcd9 * vcd9
        vrad = jnp.sqrt(sq[:, 0:3] + sq[:, 3:6] + sq[:, 6:9])

        X = jnp.concatenate(
            [jnp.concatenate(
                [vrad[:, c:c + 1], mXdb[:, c:c + 1],
                 mXdb[:, 3 + c:4 + c], mXdb[:, 6 + c:7 + c]], axis=1)
             for c in range(C)], axis=0)
        h = _leaky(_dot(X, phivw1T_ref[...]) + phivb1_ref[...])
        vef = jnp.tanh(_dot(h, phivw2T_ref[...]))
        hr = _leaky(_dot(vef, rvw1T_ref[...]) + rvb1_ref[...])
        rva = _dot(hr, rvw2T_ref[...]) + rvb2_ref[...]
        hv = _leaky(_dot(vef, vrw1T_ref[...]) + vrb1_ref[...])
        vra = _dot(hv, vrw2T_ref[...]) + vrb2_ref[...]
        rv = jnp.concatenate(
            [rva[BLKD * c:BLKD * (c + 1), :] for c in range(C)], axis=1)
        vr = jnp.concatenate(
            [vra[BLKD * c:BLKD * (c + 1), :] for c in range(C)], axis=1)

        rv9 = jnp.concatenate([rv] * 3, axis=1)
        prod = vcd9 * rv9
        vterm = -jnp.concatenate(
            [jnp.sum(prod[:, 0:3], axis=1, keepdims=True),
             jnp.sum(prod[:, 3:6], axis=1, keepdims=True),
             jnp.sum(prod[:, 6:9], axis=1, keepdims=True)], axis=1) / 3.0

        vr9 = jnp.concatenate([vr] * 3, axis=1)
        trans2 = vcd9 * vr9

        v = vel_ref[...]
        vnorm = jnp.sqrt(jnp.sum(v * v, axis=1, keepdims=True))
        hc = _leaky(vnorm * cvw1r_ref[...] + cvb1_ref[...])
        cvout = _dot(hc, cvw2T_ref[...]) + cvb2_ref[...]

        es = s0_ref[...] + s1_ref[...]
        emean = es[:, 0:3] / jnp.maximum(es[:, 3:4], 1.0)

        coord2_ref[...] = co + emean + vterm + v * cvout

        tr2p = jnp.concatenate(
            [trans2, jnp.zeros((BLKD, 7), jnp.float32)], axis=1)
        vacc_ref[...] += _dot(onehotT, tr2p)

        @pl.when(i == grid - 1)
        def _fin():
            vout_ref[...] = vc9p_ref[...] + vacc_ref[...] / cnt_g

    const = lambda shape: pl.BlockSpec(shape, lambda i: (0,) * len(shape))
    return pl.pallas_call(
        body,
        grid=(grid,),
        in_specs=[
            pl.BlockSpec((1, 1, BLKD), lambda i: (i, 0, 0)),
            pl.BlockSpec((BLKD, 1), lambda i: (i, 0)),
            pl.BlockSpec((BLKD, 3), lambda i: (i, 0)),
            pl.BlockSpec((BLKD, 3), lambda i: (i, 0)),
            pl.BlockSpec((BLKD, 8), lambda i: (i, 0)),
            pl.BlockSpec((BLKD, 8), lambda i: (i, 0)),
            const((Bp, 16)), const((Bp, 16)),
            const((4, H)), const((1, H)), const((H, H)),
            const((H, H)), const((1, H)), const((H, 1)), const((1, 1)),
            const((H, H)), const((1, H)), const((H, 1)), const((1, 1)),
            const((1, H)), const((1, H)), const((H, 1)), const((1, 1)),
        ],
        out_specs=[
            pl.BlockSpec((BLKD, 3), lambda i: (i, 0)),
            pl.BlockSpec((Bp, 16), lambda i: (0, 0)),
        ],
        out_shape=[
            jax.ShapeDtypeStruct((N, 3), jnp.float32),
            jax.ShapeDtypeStruct((Bp, 16), jnp.float32),
        ],
        scratch_shapes=[pltpu.VMEM((Bp, 16), jnp.float32)],
    )(dbT3, db2, coord, vel, s0, s1, stats, vc9p,
      phivw1T, phivb1, phivw2T,
      rvw1T, rvb1, rvw2T, rvb2,
      vrw1T, vrb1, vrw2T, vrb2,
      cvw1r, cvb1, cvw2T, cvb2)


def kernel(edge_index, data_batch, coord, node_vel, virtual_coord, edge_attr,
           phi_w1, phi_b1, phi_w2, phiv_w1, phiv_b1, phiv_w2,
           em_w1, em_b1, em_w2, em_b2, rv_w1, rv_b1, rv_w2, rv_b2,
           vr_w1, vr_b1, vr_w2, vr_b2, cv_w1, cv_b1, cv_w2, cv_b2):
    f32 = jnp.float32
    row = edge_index[0].astype(jnp.int32)
    col = edge_index[1].astype(jnp.int32)
    coordp = jnp.zeros((N, 8), f32).at[:, :3].set(coord)

    ca, cb = _sc_gather(coordp, row, col)
    ca128 = ca.reshape(E // 16, 128)
    cb128 = cb.reshape(E // 16, 128)
    ea128 = jnp.pad(edge_attr.reshape(E // 16, 16, 4),
                    ((0, 0), (0, 0), (0, 4))).reshape(E // 16, 128)
    trans128 = _edge_mlp(ca128, cb128, ea128, phi_w1, phi_b1, phi_w2,
                         em_w1, em_b1, em_w2, em_b2)
    trans = trans128.reshape(E, 8)
    zrows = jnp.zeros((ROWS_T, 8), f32)
    spart = _sc_scatter(trans, row, zrows)

    vc9p = jnp.zeros((Bp, 16), f32).at[:B, :9].set(virtual_coord.reshape(B, 9))
    dbi = data_batch.astype(jnp.int32)
    dbT3 = dbi.reshape(N // BLKD, 1, BLKD)
    db2 = dbi.reshape(N, 1)

    stats = _node_stats(dbT3, coord, vc9p)

    coord2, vout = _node_main(
        dbT3, db2, coord, node_vel, spart[0], spart[1], stats, vc9p,
        phiv_w1[:, :].T, phiv_b1.reshape(1, H), phiv_w2.T,
        rv_w1.T, rv_b1.reshape(1, H), rv_w2.T, rv_b2.reshape(1, 1),
        vr_w1.T, vr_b1.reshape(1, H), vr_w2.T, vr_b2.reshape(1, 1),
        cv_w1.reshape(1, H), cv_b1.reshape(1, H), cv_w2.T, cv_b2.reshape(1, 1))

    virtual_coord2 = vout[:B, :9].reshape(B, 3, C)
    return coord2, virtual_coord2
